# baseline (speedup 1.0000x reference)
import jax
import jax.numpy as jnp
from jax.experimental import pallas as pl
from jax.experimental.pallas import tpu as pltpu

_NX, _NY, _NZ = 432, 496, 1
_C = 64
_P = 100
_COLS = _NZ * _NY * _NX
_W = 6912
_NBLK = _COLS // _W


def _scatter_kernel(flags_ref, coords_ref, feats_ref, out_ref):
    b = pl.program_id(0)

    @pl.when(flags_ref[b] == 0)
    def _zero():
        out_ref[...] = jnp.zeros_like(out_ref)

    @pl.when(flags_ref[b] != 0)
    def _scatter():
        coords = coords_ref[...]
        idx = coords[:, 1:2] + coords[:, 2:3] * _NX + coords[:, 3:4]
        base = b * _W
        cols = jax.lax.broadcasted_iota(jnp.int32, (_P, _W), 1) + base
        onehot = (idx == cols).astype(jnp.float32)
        feats = feats_ref[...]
        blk = jax.lax.dot_general(
            feats, onehot, (((0,), (0,)), ((), ())),
            preferred_element_type=jnp.float32)
        out_ref[...] = blk


def kernel(pillar_features, voxel_coords):
    coords = voxel_coords.astype(jnp.int32)
    indices = coords[:, 1] + coords[:, 2] * _NX + coords[:, 3]
    flags = jnp.zeros((_NBLK,), jnp.int32).at[indices // _W].set(1, mode="drop")

    grid_spec = pltpu.PrefetchScalarGridSpec(
        num_scalar_prefetch=1,
        grid=(_NBLK,),
        in_specs=[
            pl.BlockSpec((_P, 4), lambda b, flags: (0, 0)),
            pl.BlockSpec((_P, _C), lambda b, flags: (0, 0)),
        ],
        out_specs=pl.BlockSpec((_C, _W), lambda b, flags: (0, b)),
    )
    out = pl.pallas_call(
        _scatter_kernel,
        grid_spec=grid_spec,
        out_shape=jax.ShapeDtypeStruct((_C, _COLS), jnp.float32),
        compiler_params=pltpu.CompilerParams(
            dimension_semantics=("arbitrary",)),
    )(flags, coords, pillar_features[:_P, :])
    return out.reshape(1, _C * _NZ, _NY, _NX)


# speedup vs baseline: 4.3046x; 4.3046x over previous
import jax
import jax.numpy as jnp
from jax.experimental import pallas as pl
from jax.experimental.pallas import tpu as pltpu

_NX, _NY, _NZ = 432, 496, 1
_C = 64
_P = 100
_ROWS = 16
_NBLK = _NY // _ROWS


def _scatter_kernel(flags_ref, coords_ref, feats_ref, out_ref):
    b = pl.program_id(0)

    @pl.when(flags_ref[b] == 0)
    def _zero():
        out_ref[...] = jnp.zeros_like(out_ref)

    @pl.when(flags_ref[b] != 0)
    def _scatter():
        coords = coords_ref[...]
        idx = coords[:, 1:2] + coords[:, 2:3] * _NX + coords[:, 3:4]
        feats = feats_ref[...]
        for r in range(_ROWS):
            y = b * _ROWS + r
            cols = jax.lax.broadcasted_iota(jnp.int32, (_P, _NX), 1) + y * _NX
            onehot = (idx == cols).astype(jnp.float32)
            row = jax.lax.dot_general(
                feats, onehot, (((0,), (0,)), ((), ())),
                preferred_element_type=jnp.float32)
            out_ref[0, :, r, :] = row


def kernel(pillar_features, voxel_coords):
    coords = voxel_coords.astype(jnp.int32)
    indices = coords[:, 1] + coords[:, 2] * _NX + coords[:, 3]
    flags = jnp.zeros((_NBLK,), jnp.int32).at[indices // (_NX * _ROWS)].set(
        1, mode="drop")

    grid_spec = pltpu.PrefetchScalarGridSpec(
        num_scalar_prefetch=1,
        grid=(_NBLK,),
        in_specs=[
            pl.BlockSpec((_P, 4), lambda b, flags: (0, 0)),
            pl.BlockSpec((_P, _C), lambda b, flags: (0, 0)),
        ],
        out_specs=pl.BlockSpec((1, _C, _ROWS, _NX), lambda b, flags: (0, 0, b, 0)),
    )
    out = pl.pallas_call(
        _scatter_kernel,
        grid_spec=grid_spec,
        out_shape=jax.ShapeDtypeStruct((1, _C * _NZ, _NY, _NX), jnp.float32),
        compiler_params=pltpu.CompilerParams(
            dimension_semantics=("arbitrary",)),
    )(flags, coords, pillar_features[:_P, :])
    return out


# speedup vs baseline: 4.3456x; 1.0095x over previous
import jax
import jax.numpy as jnp
from jax.experimental import pallas as pl
from jax.experimental.pallas import tpu as pltpu

_NX, _NY, _NZ = 432, 496, 1
_C = 64
_P = 100
_R = 64
_CHUNK_STARTS = list(range(0, _NY - _NY % _R, _R))
_LAST_START = _CHUNK_STARTS[-1] + _R
_LAST_ROWS = _NY - _LAST_START
_T = 8


def _scatter_kernel(tileids_ref, ntiles_ref, coords_ref, feats_ref, out_ref,
                    zbuf, sbuf, zsem, ssem):
    zbuf[...] = jnp.zeros_like(zbuf)
    copies = [
        pltpu.make_async_copy(
            zbuf, out_ref.at[:, :, pl.ds(s, _R), :], zsem)
        for s in _CHUNK_STARTS
    ]
    copies.append(pltpu.make_async_copy(
        zbuf.at[:, :, pl.ds(0, _LAST_ROWS), :],
        out_ref.at[:, :, pl.ds(_LAST_START, _LAST_ROWS), :], zsem))
    for c in copies:
        c.start()
    for c in copies:
        c.wait()

    coords = coords_ref[...]
    idx = coords[:, 1:2] + coords[:, 2:3] * _NX + coords[:, 3:4]
    feats = feats_ref[...]

    def body(i, carry):
        t = tileids_ref[i]
        for r in range(_T):
            y = t * _T + r
            cols = jax.lax.broadcasted_iota(jnp.int32, (_P, _NX), 1) + y * _NX
            onehot = (idx == cols).astype(jnp.float32)
            row = jax.lax.dot_general(
                feats, onehot, (((0,), (0,)), ((), ())),
                preferred_element_type=jnp.float32)
            sbuf[0, :, r, :] = row
        cp = pltpu.make_async_copy(
            sbuf, out_ref.at[:, :, pl.ds(t * _T, _T), :], ssem)
        cp.start()
        cp.wait()
        return carry

    jax.lax.fori_loop(0, ntiles_ref[0], body, 0)


def kernel(pillar_features, voxel_coords):
    coords = voxel_coords.astype(jnp.int32)
    indices = coords[:, 1] + coords[:, 2] * _NX + coords[:, 3]
    tiles = indices // (_NX * _T)
    dup = jnp.tril(tiles[None, :] == tiles[:, None], k=-1).any(axis=1)
    keep = ~dup
    pos = jnp.cumsum(keep.astype(jnp.int32)) - 1
    tileids = jnp.zeros((_P,), jnp.int32).at[
        jnp.where(keep, pos, _P)].set(tiles, mode="drop")
    ntiles = keep.sum(dtype=jnp.int32).reshape(1)

    grid_spec = pltpu.PrefetchScalarGridSpec(
        num_scalar_prefetch=2,
        grid=(1,),
        in_specs=[
            pl.BlockSpec((_P, 4), lambda i, *_: (0, 0)),
            pl.BlockSpec((_P, _C), lambda i, *_: (0, 0)),
        ],
        out_specs=pl.BlockSpec(memory_space=pltpu.MemorySpace.HBM),
        scratch_shapes=[
            pltpu.VMEM((1, _C, _R, _NX), jnp.float32),
            pltpu.VMEM((1, _C, _T, _NX), jnp.float32),
            pltpu.SemaphoreType.DMA,
            pltpu.SemaphoreType.DMA,
        ],
    )
    out = pl.pallas_call(
        _scatter_kernel,
        grid_spec=grid_spec,
        out_shape=jax.ShapeDtypeStruct((1, _C * _NZ, _NY, _NX), jnp.float32),
    )(tileids, ntiles, coords, pillar_features[:_P, :])
    return out


# speedup vs baseline: 4.3533x; 1.0018x over previous
import jax
import jax.numpy as jnp
from jax.experimental import pallas as pl
from jax.experimental.pallas import tpu as pltpu

_NX, _NY, _NZ = 432, 496, 1
_C = 64
_P = 100
_CC = 8
_NCHUNK = _C // _CC
_T = 8


def _scatter_kernel(tileids_ref, ntiles_ref, coords_ref, feats_ref, out_ref,
                    zbuf, sbuf, zsem, ssem):
    zbuf[...] = jnp.zeros_like(zbuf)
    copies = [
        pltpu.make_async_copy(
            zbuf, out_ref.at[:, pl.ds(k * _CC, _CC), :, :], zsem)
        for k in range(_NCHUNK)
    ]
    for c in copies:
        c.start()
    for c in copies:
        c.wait()

    coords = coords_ref[...]
    idx = coords[:, 1:2] + coords[:, 2:3] * _NX + coords[:, 3:4]
    feats = feats_ref[...]

    def body(i, carry):
        t = tileids_ref[i]
        for r in range(_T):
            y = t * _T + r
            cols = jax.lax.broadcasted_iota(jnp.int32, (_P, _NX), 1) + y * _NX
            onehot = (idx == cols).astype(jnp.float32)
            row = jax.lax.dot_general(
                feats, onehot, (((0,), (0,)), ((), ())),
                preferred_element_type=jnp.float32)
            sbuf[0, :, r, :] = row
        cp = pltpu.make_async_copy(
            sbuf, out_ref.at[:, :, pl.ds(t * _T, _T), :], ssem)
        cp.start()
        cp.wait()
        return carry

    jax.lax.fori_loop(0, ntiles_ref[0], body, 0)


def kernel(pillar_features, voxel_coords):
    coords = voxel_coords.astype(jnp.int32)
    indices = coords[:, 1] + coords[:, 2] * _NX + coords[:, 3]
    tiles = indices // (_NX * _T)
    dup = jnp.tril(tiles[None, :] == tiles[:, None], k=-1).any(axis=1)
    keep = ~dup
    pos = jnp.cumsum(keep.astype(jnp.int32)) - 1
    tileids = jnp.zeros((_P,), jnp.int32).at[
        jnp.where(keep, pos, _P)].set(tiles, mode="drop")
    ntiles = keep.sum(dtype=jnp.int32).reshape(1)

    grid_spec = pltpu.PrefetchScalarGridSpec(
        num_scalar_prefetch=2,
        grid=(1,),
        in_specs=[
            pl.BlockSpec((_P, 4), lambda i, *_: (0, 0)),
            pl.BlockSpec((_P, _C), lambda i, *_: (0, 0)),
        ],
        out_specs=pl.BlockSpec(memory_space=pltpu.MemorySpace.HBM),
        scratch_shapes=[
            pltpu.VMEM((1, _CC, _NY, _NX), jnp.float32),
            pltpu.VMEM((1, _C, _T, _NX), jnp.float32),
            pltpu.SemaphoreType.DMA,
            pltpu.SemaphoreType.DMA,
        ],
    )
    out = pl.pallas_call(
        _scatter_kernel,
        grid_spec=grid_spec,
        out_shape=jax.ShapeDtypeStruct((1, _C * _NZ, _NY, _NX), jnp.float32),
    )(tileids, ntiles, coords, pillar_features[:_P, :])
    return out
